# baseline (speedup 1.0000x reference)
import jax
import jax.numpy as jnp
from jax import lax
from jax.experimental import pallas as pl
from jax.experimental.pallas import tpu as pltpu
from jax.experimental.pallas import tpu_sc as plsc

B, P, H, W = 4096, 2, 24, 24
C = 7
PH = P * H
W8 = W // 8
NB = B // 128
NW = 32
NT = 4
NR = NT * 8
ROWS = PH * W8 * NB * 8
UNITS = PH * W8 * (NB // NT)
UPW = UNITS // NW
NQ = NB // NT
L = 16


def _as_rows(x):
    x = jnp.transpose(x, (1, 2, 3, 0))
    x = x.reshape(P, H, W8, 8, NB, 128)
    x = jnp.transpose(x, (0, 1, 2, 4, 3, 5))
    return x.reshape(ROWS, 128)


def _in_row(u):
    return (u // NQ * NB + u % NQ * NT) * 8


def _body(tt_hbm, uc_hbm, rm_hbm, rw_hbm, tab_hbm, out_hbm,
          tt_v, rm_v, rw_v, tab_v, out_v, rv_slab, in_sems, out_sems, uc_sems):
    wid = lax.axis_index("s") * 2 + lax.axis_index("c")
    pltpu.sync_copy(tab_hbm, tab_v)
    pltpu.sync_copy(rw_hbm, rw_v)
    start = wid * UPW
    end = start + UPW
    p = wid // 16

    def _prefetch(u, b):
        r = _in_row(u)
        pltpu.async_copy(tt_hbm.at[pl.ds(r, NR), :], tt_v.at[b], in_sems.at[b])
        pltpu.async_copy(rm_hbm.at[pl.ds(r, NR), :], rm_v.at[b], in_sems.at[b])

    _prefetch(start, 0)
    _prefetch(start + 1, 1)
    pltpu.async_copy(uc_hbm.at[pl.ds(_in_row(start), NR), :], out_v.at[0, 4],
                     uc_sems.at[0])
    pltpu.async_copy(uc_hbm.at[pl.ds(_in_row(start + 1), NR), :],
                     out_v.at[1, 4], uc_sems.at[1])

    for q in range(NQ):
        for k in range(NT):
            for j in range(8):
                val = rw_v[pl.ds(q * NT * P * 128 + k * P * 128 + p * 128
                                 + j * L, L)]
                for sub in range(8):
                    rv_slab[q, k * 8 + sub, pl.ds(j * L, L)] = val

    def out_slice6(u):
        q = u % NQ
        row = u // NQ
        ph = row // W8
        w8 = row % W8
        rr = (w8 * NB + q * NT) * 8
        return (out_hbm.at[ph, pl.ds(0, 6), pl.ds(rr, NR), :],
                out_hbm.at[ph, 6, pl.ds(rr, NR), :])

    def start_in(u, b):
        r = _in_row(u)
        pltpu.async_copy(tt_hbm.at[pl.ds(r, NR), :], tt_v.at[b], in_sems.at[b])
        pltpu.async_copy(rm_hbm.at[pl.ds(r, NR), :], rm_v.at[b], in_sems.at[b])

    def wait_in(u, b):
        r = _in_row(u)
        pltpu.make_async_copy(
            tt_hbm.at[pl.ds(r, NR), :], tt_v.at[b], in_sems.at[b]).wait()
        pltpu.make_async_copy(
            rm_hbm.at[pl.ds(r, NR), :], rm_v.at[b], in_sems.at[b]).wait()

    def start_uc(u, b):
        pltpu.async_copy(uc_hbm.at[pl.ds(_in_row(u), NR), :], out_v.at[b, 4],
                         uc_sems.at[b])

    def wait_uc(u, b):
        pltpu.make_async_copy(
            uc_hbm.at[pl.ds(_in_row(u), NR), :], out_v.at[b, 4],
            uc_sems.at[b]).wait()

    def start_out(u, b):
        d6, dr = out_slice6(u)
        pltpu.async_copy(out_v.at[b], d6, out_sems.at[b])
        pltpu.async_copy(rv_slab.at[u % NQ], dr, out_sems.at[b])

    def wait_out(u, b):
        d6, dr = out_slice6(u)
        pltpu.make_async_copy(out_v.at[b], d6, out_sems.at[b]).wait()
        pltpu.make_async_copy(rv_slab.at[u % NQ], dr, out_sems.at[b]).wait()

    tvec = tab_v[pl.ds(0, L)]
    tb = [[jnp.full((L,), tvec[c * 4 + r], jnp.float32) for r in range(4)]
          for c in range(4)]

    def compute(u, b):
        for row in range(NR):

            @plsc.parallel_loop(0, 128, L, unroll=8)
            def lane_body(o16, row=row, b=b):
                sl = pl.ds(o16, L)
                tt = tt_v[b, row, sl]
                rm = rm_v[b, row, sl]
                m1 = tt >= 1
                m2 = tt >= 2
                m3 = tt >= 3
                for c in range(4):
                    e = jnp.where(m1, tb[c][1], tb[c][0])
                    e = jnp.where(m2, tb[c][2], e)
                    e = jnp.where(m3, tb[c][3], e)
                    out_v[b, c, row, sl] = e
                out_v[b, 5, row, sl] = rm.astype(jnp.float32)

    def pair_body(t, carry):
        g0 = start + 2 * t
        for b in range(2):
            u = g0 + b

            wait_in(u, b)
            compute(u, b)
            wait_uc(u, b)

            @pl.when(jnp.logical_and(u >= start + 1, u + 1 < end))
            def _(u=u, b=b):
                wait_out(u - 1, 1 - b)
                start_uc(u + 1, 1 - b)

            start_out(u, b)

            @pl.when(u + 2 < end)
            def _(u=u, b=b):
                start_in(u + 2, b)

        return carry

    lax.fori_loop(0, UPW // 2, pair_body, 0)
    wait_out(end - 2, 0)
    wait_out(end - 1, 1)


def kernel(tile_type, unit_counts_player_0, relic_map,
           normalized_reward_last_round, embed_table):
    tt = _as_rows(tile_type)
    uc = _as_rows(unit_counts_player_0)
    rm = _as_rows(relic_map)
    rw = normalized_reward_last_round.reshape(NB, 128, P)
    rw = jnp.transpose(rw, (0, 2, 1)).reshape(NB * P * 128)
    tab = embed_table.T.reshape(L)

    mesh = plsc.VectorSubcoreMesh(core_axis_name="c", subcore_axis_name="s")
    run = pl.kernel(
        _body,
        mesh=mesh,
        compiler_params=pltpu.CompilerParams(needs_layout_passes=False),
        out_type=jax.ShapeDtypeStruct((PH, C, W8 * NB * 8, 128), jnp.float32),
        scratch_types=[
            pltpu.VMEM((2, NR, 128), jnp.int32),
            pltpu.VMEM((2, NR, 128), jnp.int32),
            pltpu.VMEM((NB * P * 128,), jnp.float32),
            pltpu.VMEM((L,), jnp.float32),
            pltpu.VMEM((2, 6, NR, 128), jnp.float32),
            pltpu.VMEM((NQ, NR, 128), jnp.float32),
            pltpu.SemaphoreType.DMA((2,)),
            pltpu.SemaphoreType.DMA((2,)),
            pltpu.SemaphoreType.DMA((2,)),
        ],
    )
    out = run(tt, uc, rm, rw, tab)
    out = out.reshape(P, H, C, W8, NB, 8, 128)
    out = out.transpose(4, 6, 0, 1, 3, 5, 2)
    return out.reshape(B, P, H, W, C)


# speedup vs baseline: 1.0124x; 1.0124x over previous
import jax
import jax.numpy as jnp
from jax import lax
from jax.experimental import pallas as pl
from jax.experimental.pallas import tpu as pltpu
from jax.experimental.pallas import tpu_sc as plsc

B, P, H, W = 4096, 2, 24, 24
C = 7
PH = P * H
W8 = W // 8
NB = B // 128
NW = 32
NT = 4
NR = NT * 8
ROWS = PH * W8 * NB * 8
UNITS = PH * W8 * (NB // NT)
UPW = UNITS // NW
NQ = NB // NT
L = 16


def _as_rows(x):
    x = jnp.transpose(x, (1, 2, 3, 0))
    x = x.reshape(P, H, W8, 8, NB, 128)
    x = jnp.transpose(x, (0, 1, 2, 4, 3, 5))
    return x.reshape(ROWS, 128)


def _in_row(u):
    return (u // NQ * NB + u % NQ * NT) * 8


def _body(tt_hbm, uc_hbm, rm_hbm, rw_hbm, tab_hbm, out_hbm,
          tt_v, rm_v, rw_v, tab_v, out_v, rv_slab, in_sems, out_sems, uc_sems):
    wid = lax.axis_index("s") * 2 + lax.axis_index("c")
    pltpu.sync_copy(tab_hbm, tab_v)
    pltpu.sync_copy(rw_hbm, rw_v)
    start = wid * UPW
    end = start + UPW
    p = wid // 16

    def _prefetch(u, b):
        r = _in_row(u)
        pltpu.async_copy(tt_hbm.at[pl.ds(r, NR), :], tt_v.at[b], in_sems.at[b])
        pltpu.async_copy(rm_hbm.at[pl.ds(r, NR), :], rm_v.at[b], in_sems.at[b])

    _prefetch(start, 0)
    _prefetch(start + 1, 1)

    for q in range(NQ):
        for k in range(NT):
            for j in range(8):
                val = rw_v[pl.ds(q * NT * P * 128 + k * P * 128 + p * 128
                                 + j * L, L)]
                for sub in range(8):
                    rv_slab[q, k * 8 + sub, pl.ds(j * L, L)] = val

    def out_slice6(u):
        q = u % NQ
        row = u // NQ
        ph = row // W8
        w8 = row % W8
        rr = (w8 * NB + q * NT) * 8
        return (out_hbm.at[ph, pl.ds(0, 6), pl.ds(rr, NR), :],
                out_hbm.at[ph, 6, pl.ds(rr, NR), :])

    def start_in(u, b):
        r = _in_row(u)
        pltpu.async_copy(tt_hbm.at[pl.ds(r, NR), :], tt_v.at[b], in_sems.at[b])
        pltpu.async_copy(rm_hbm.at[pl.ds(r, NR), :], rm_v.at[b], in_sems.at[b])

    def wait_in(u, b):
        r = _in_row(u)
        pltpu.make_async_copy(
            tt_hbm.at[pl.ds(r, NR), :], tt_v.at[b], in_sems.at[b]).wait()
        pltpu.make_async_copy(
            rm_hbm.at[pl.ds(r, NR), :], rm_v.at[b], in_sems.at[b]).wait()

    def start_uc(u, b):
        pltpu.async_copy(uc_hbm.at[pl.ds(_in_row(u), NR), :], out_v.at[b, 4],
                         uc_sems.at[b])

    def wait_uc(u, b):
        pltpu.make_async_copy(
            uc_hbm.at[pl.ds(_in_row(u), NR), :], out_v.at[b, 4],
            uc_sems.at[b]).wait()

    def start_out(u, b):
        d6, dr = out_slice6(u)
        pltpu.async_copy(out_v.at[b], d6, out_sems.at[b])
        pltpu.async_copy(rv_slab.at[u % NQ], dr, out_sems.at[b])

    def wait_out(u, b):
        d6, dr = out_slice6(u)
        pltpu.make_async_copy(out_v.at[b], d6, out_sems.at[b]).wait()
        pltpu.make_async_copy(rv_slab.at[u % NQ], dr, out_sems.at[b]).wait()

    tvec = tab_v[pl.ds(0, L)]
    tb = [[jnp.full((L,), tvec[c * 4 + r], jnp.float32) for r in range(4)]
          for c in range(4)]

    def compute(u, b):
        for row in range(NR):

            @plsc.parallel_loop(0, 128, L, unroll=8)
            def lane_body(o16, row=row, b=b):
                sl = pl.ds(o16, L)
                tt = tt_v[b, row, sl]
                rm = rm_v[b, row, sl]
                m1 = tt >= 1
                m2 = tt >= 2
                m3 = tt >= 3
                for c in range(4):
                    e = jnp.where(m1, tb[c][1], tb[c][0])
                    e = jnp.where(m2, tb[c][2], e)
                    e = jnp.where(m3, tb[c][3], e)
                    out_v[b, c, row, sl] = e
                out_v[b, 5, row, sl] = rm.astype(jnp.float32)

    def pair_body(t, carry):
        g0 = start + 2 * t
        for b in range(2):
            u = g0 + b

            @pl.when(u >= start + 2)
            def _(u=u, b=b):
                wait_out(u - 2, b)

            start_uc(u, b)
            wait_in(u, b)
            compute(u, b)
            wait_uc(u, b)
            start_out(u, b)

            @pl.when(u + 2 < end)
            def _(u=u, b=b):
                start_in(u + 2, b)

        return carry

    lax.fori_loop(0, UPW // 2, pair_body, 0)
    wait_out(end - 2, 0)
    wait_out(end - 1, 1)


def kernel(tile_type, unit_counts_player_0, relic_map,
           normalized_reward_last_round, embed_table):
    tt = _as_rows(tile_type)
    uc = _as_rows(unit_counts_player_0)
    rm = _as_rows(relic_map)
    rw = normalized_reward_last_round.reshape(NB, 128, P)
    rw = jnp.transpose(rw, (0, 2, 1)).reshape(NB * P * 128)
    tab = embed_table.T.reshape(L)

    mesh = plsc.VectorSubcoreMesh(core_axis_name="c", subcore_axis_name="s")
    run = pl.kernel(
        _body,
        mesh=mesh,
        compiler_params=pltpu.CompilerParams(needs_layout_passes=False),
        out_type=jax.ShapeDtypeStruct((PH, C, W8 * NB * 8, 128), jnp.float32),
        scratch_types=[
            pltpu.VMEM((2, NR, 128), jnp.int32),
            pltpu.VMEM((2, NR, 128), jnp.int32),
            pltpu.VMEM((NB * P * 128,), jnp.float32),
            pltpu.VMEM((L,), jnp.float32),
            pltpu.VMEM((2, 6, NR, 128), jnp.float32),
            pltpu.VMEM((NQ, NR, 128), jnp.float32),
            pltpu.SemaphoreType.DMA((2,)),
            pltpu.SemaphoreType.DMA((2,)),
            pltpu.SemaphoreType.DMA((2,)),
        ],
    )
    out = run(tt, uc, rm, rw, tab)
    out = out.reshape(P, H, C, W8, NB, 8, 128)
    out = out.transpose(4, 6, 0, 1, 3, 5, 2)
    return out.reshape(B, P, H, W, C)
